# Initial kernel scaffold; baseline (speedup 1.0000x reference)
import jax
import jax.numpy as jnp
from jax.experimental import pallas as pl


def kernel(input_ids, emb, n1w, n1b, n2w, n2b, in_w, in_b, out_w, out_b, rw, rb, f1w, f1b, f2w, f2b, lm_w, lm_b):
    raise NotImplementedError("write your pallas kernel here")


# speedup vs baseline: 1.6388x; 1.6388x over previous
import functools
import math

import numpy as _np

import jax
import jax.numpy as jnp
from jax import lax
from jax.experimental import pallas as pl
from jax.experimental.pallas import tpu as pltpu
from jax.experimental.pallas import tpu_sc as plsc

_CONTRACT_T = (((1,), (1,)), ((), ()))


def _bf(x):
    return x.astype(jnp.bfloat16)


def _embed_gather(emb, ids):
    S = ids.shape[0]
    D = emb.shape[1]
    info = plsc.get_sparse_core_info()
    nw = info.num_cores * info.num_subcores
    b_per_w = S // nw
    mesh = plsc.VectorSubcoreMesh(core_axis_name="c", subcore_axis_name="s")

    @functools.partial(
        pl.kernel,
        mesh=mesh,
        out_type=jax.ShapeDtypeStruct((S, D), jnp.float32),
        scratch_types=[
            pltpu.VMEM((b_per_w,), jnp.int32),
            pltpu.VMEM((b_per_w, D), jnp.float32),
            pltpu.SemaphoreType.DMA,
        ],
    )
    def k(emb_hbm, idx_hbm, out_hbm, idx_v, rows_v, sem):
        wid = lax.axis_index("s") * info.num_cores + lax.axis_index("c")
        base = wid * b_per_w
        pltpu.sync_copy(idx_hbm.at[pl.ds(base, b_per_w)], idx_v)
        pltpu.async_copy(emb_hbm.at[idx_v], rows_v, sem).wait()
        pltpu.sync_copy(rows_v, out_hbm.at[pl.ds(base, b_per_w)])

    return k(emb, ids)


def _gelu(x):
    return 0.5 * x * (1.0 + lax.erf(x * (1.0 / math.sqrt(2.0))))


def _moe_body(nx_ref, sel_ref, w1_ref, b1_ref, w2_ref, b2_ref, hres_ref,
              o_ref, cnt_ref, aux_ref, *, tb, nE, nF, nT, target):
    e = pl.program_id(0)
    fc = pl.program_id(1)
    t = pl.program_id(2)

    @pl.when((e == 0) & (fc == 0) & (t == 0))
    def _():
        o_ref[...] = hres_ref[...]
        cnt_ref[...] = jnp.zeros_like(cnt_ref)

    sel = sel_ref[...]

    @pl.when((e == 0) & (fc == 0))
    def _():
        iota8 = lax.broadcasted_iota(jnp.int32, (1, cnt_ref.shape[1]), 1)
        c = jnp.sum((sel[:, 0:1] == iota8).astype(jnp.float32), axis=0,
                    keepdims=True)
        c += jnp.sum((sel[:, 1:2] == iota8).astype(jnp.float32), axis=0,
                     keepdims=True)
        cnt_ref[...] += c

        @pl.when(t == nT - 1)
        def _():
            d = cnt_ref[...] - target
            aux_ref[...] = (jnp.sum(d * d, axis=1, keepdims=True)
                            * (0.01 / cnt_ref.shape[1]))

    x = _bf(nx_ref[...])
    hid = lax.dot_general(x, _bf(w1_ref[0, 0]), _CONTRACT_T,
                          preferred_element_type=jnp.float32) + b1_ref[0]
    hid = _gelu(hid)
    y = lax.dot_general(_bf(hid), _bf(w2_ref[0, 0]), _CONTRACT_T,
                        preferred_element_type=jnp.float32)
    y = y + jnp.where(fc == 0, 1.0, 0.0) * b2_ref[0]
    mcol = jnp.sum((sel == e).astype(jnp.float32), axis=1, keepdims=True)
    o_ref[pl.ds(t * tb, tb), :] += mcol * y


def _moe(nx, sel, f1w, f1b, f2w, f2b, hres, tb, fchunks):
    S, D = nx.shape
    E = f1w.shape[1]
    F = f1w.shape[2]
    fcs = F // fchunks
    f1b = f1b.reshape(E, 1, F)
    f2b = f2b.reshape(E, 1, D)
    return pl.pallas_call(
        functools.partial(_moe_body, tb=tb, nE=E, nF=fchunks, nT=S // tb,
                          target=S / E),
        grid=(E, fchunks, S // tb),
        in_specs=[
            pl.BlockSpec((tb, D), lambda e, f, t: (t, 0)),
            pl.BlockSpec((tb, 2), lambda e, f, t: (t, 0)),
            pl.BlockSpec((1, 1, fcs, D), lambda e, f, t: (0, e, f, 0)),
            pl.BlockSpec((1, 1, fcs), lambda e, f, t: (e, 0, f)),
            pl.BlockSpec((1, 1, D, fcs), lambda e, f, t: (0, e, 0, f)),
            pl.BlockSpec((1, 1, D), lambda e, f, t: (e, 0, 0)),
            pl.BlockSpec((S, D), lambda e, f, t: (0, 0)),
        ],
        out_specs=[
            pl.BlockSpec((S, D), lambda e, f, t: (0, 0)),
            pl.BlockSpec((1, E), lambda e, f, t: (0, 0)),
            pl.BlockSpec((1, 1), lambda e, f, t: (0, 0)),
        ],
        out_shape=[
            jax.ShapeDtypeStruct((S, D), jnp.float32),
            jax.ShapeDtypeStruct((1, E), jnp.float32),
            jax.ShapeDtypeStruct((1, 1), jnp.float32),
        ],
    )(nx, sel, f1w, f1b, f2w, f2b, hres)


def _lm_body(h_ref, w_ref, b_ref, o_ref, *, tb):
    t = pl.program_id(1)
    x = _bf(h_ref[pl.ds(t * tb, tb), :])
    o_ref[...] = (
        lax.dot_general(x, _bf(w_ref[...]), _CONTRACT_T,
                        preferred_element_type=jnp.float32)
        + b_ref[...]
    )


def _lm_head(h, w, b, tb, vb):
    S, D = h.shape
    V = w.shape[0]
    return pl.pallas_call(
        functools.partial(_lm_body, tb=tb),
        grid=(V // vb, S // tb),
        in_specs=[
            pl.BlockSpec((S, D), lambda v, t: (0, 0)),
            pl.BlockSpec((vb, D), lambda v, t: (v, 0)),
            pl.BlockSpec((1, vb), lambda v, t: (0, v)),
        ],
        out_specs=pl.BlockSpec((tb, vb), lambda v, t: (t, v)),
        out_shape=jax.ShapeDtypeStruct((S, V), jnp.float32),
    )(h, w, b)


def kernel(input_ids, emb, n1w, n1b, n2w, n2b, in_w, in_b, out_w, out_b,
           rw, rb, f1w, f1b, f2w, f2b, lm_w, lm_b):
    Bz, S = input_ids.shape
    V, D = emb.shape
    H = 8
    hd = D // H

    ids = input_ids.reshape(-1).astype(jnp.int32)

    h0 = _embed_gather(emb, ids)

    def _ln(x, w, b):
        m = x.mean(-1, keepdims=True)
        v = x.var(-1, keepdims=True)
        return (x - m) / jnp.sqrt(v + 1e-5) * w + b

    h = h0.reshape(Bz, S, D)
    nx = _ln(h, n1w[0], n1b[0])
    qkv = nx @ in_w[0].T + in_b[0]
    q, k, v = jnp.split(qkv, 3, axis=-1)

    def sp(z):
        return z.reshape(Bz, S, H, hd).transpose(0, 2, 1, 3)

    q, k, v = sp(q), sp(k), sp(v)
    a = jax.nn.softmax(
        (q @ k.transpose(0, 1, 3, 2)) / jnp.sqrt(jnp.float32(hd)), axis=-1)
    o = (a @ v).transpose(0, 2, 1, 3).reshape(Bz, S, D)
    h = h + o @ out_w[0].T + out_b[0]
    nx2 = _ln(h, n2w[0], n2b[0])
    flat = nx2.reshape(-1, D)
    rlog = flat @ rw[0].T + rb[0]
    _, sel = jax.lax.top_k(rlog, 2)

    hfin, _cnt, aux = _moe(flat, sel, f1w, f1b, f2w, f2b,
                           h.reshape(S, D), 256, fchunks=2)
    logits = _lm_head(hfin, lm_w, lm_b.reshape(1, V), 256, 3200)

    return logits.reshape(Bz, S, V), aux[0, 0]


# speedup vs baseline: 1.7961x; 1.0960x over previous
import functools
import math

import numpy as _np

import jax
import jax.numpy as jnp
from jax import lax
from jax.experimental import pallas as pl
from jax.experimental.pallas import tpu as pltpu
from jax.experimental.pallas import tpu_sc as plsc

_CONTRACT_T = (((1,), (1,)), ((), ()))


def _bf(x):
    return x.astype(jnp.bfloat16)


def _embed_gather(emb, ids):
    S = ids.shape[0]
    D = emb.shape[1]
    info = plsc.get_sparse_core_info()
    nw = info.num_cores * info.num_subcores
    b_per_w = S // nw
    mesh = plsc.VectorSubcoreMesh(core_axis_name="c", subcore_axis_name="s")

    @functools.partial(
        pl.kernel,
        mesh=mesh,
        out_type=jax.ShapeDtypeStruct((S, D), jnp.float32),
        scratch_types=[
            pltpu.VMEM((b_per_w,), jnp.int32),
            pltpu.VMEM((b_per_w, D), jnp.float32),
            pltpu.SemaphoreType.DMA,
        ],
    )
    def k(emb_hbm, idx_hbm, out_hbm, idx_v, rows_v, sem):
        wid = lax.axis_index("s") * info.num_cores + lax.axis_index("c")
        base = wid * b_per_w
        pltpu.sync_copy(idx_hbm.at[pl.ds(base, b_per_w)], idx_v)
        pltpu.async_copy(emb_hbm.at[idx_v], rows_v, sem).wait()
        pltpu.sync_copy(rows_v, out_hbm.at[pl.ds(base, b_per_w)])

    return k(emb, ids)


def _gelu(x):
    return 0.5 * x * (1.0 + lax.erf(x * (1.0 / math.sqrt(2.0))))


def _moe_body(nx_ref, sel_ref, w1_ref, b1_ref, w2_ref, b2_ref, hres_ref,
              o_ref, cnt_ref, aux_ref, *, tb, nE, nF, nT, target):
    e = pl.program_id(0)
    fc = pl.program_id(1)
    t = pl.program_id(2)

    @pl.when((e == 0) & (fc == 0) & (t == 0))
    def _():
        o_ref[...] = hres_ref[...]
        cnt_ref[...] = jnp.zeros_like(cnt_ref)

    sel = sel_ref[...]

    @pl.when((e == 0) & (fc == 0))
    def _():
        iota8 = lax.broadcasted_iota(jnp.int32, (1, cnt_ref.shape[1]), 1)
        c = jnp.sum((sel[:, 0:1] == iota8).astype(jnp.float32), axis=0,
                    keepdims=True)
        c += jnp.sum((sel[:, 1:2] == iota8).astype(jnp.float32), axis=0,
                     keepdims=True)
        cnt_ref[...] += c

        @pl.when(t == nT - 1)
        def _():
            d = cnt_ref[...] - target
            aux_ref[...] = (jnp.sum(d * d, axis=1, keepdims=True)
                            * (0.01 / cnt_ref.shape[1]))

    x = _bf(nx_ref[...])
    hid = lax.dot_general(x, _bf(w1_ref[0, 0]), _CONTRACT_T,
                          preferred_element_type=jnp.float32) + b1_ref[0]
    hid = _gelu(hid)
    y = lax.dot_general(_bf(hid), _bf(w2_ref[0, 0]), _CONTRACT_T,
                        preferred_element_type=jnp.float32)
    y = y + jnp.where(fc == 0, 1.0, 0.0) * b2_ref[0]
    mcol = jnp.sum((sel == e).astype(jnp.float32), axis=1, keepdims=True)
    o_ref[pl.ds(t * tb, tb), :] += mcol * y


def _moe(nx, sel, f1w, f1b, f2w, f2b, hres, tb, fchunks):
    S, D = nx.shape
    E = f1w.shape[1]
    F = f1w.shape[2]
    fcs = F // fchunks
    f1b = f1b.reshape(E, 1, F)
    f2b = f2b.reshape(E, 1, D)
    return pl.pallas_call(
        functools.partial(_moe_body, tb=tb, nE=E, nF=fchunks, nT=S // tb,
                          target=S / E),
        grid=(E, fchunks, S // tb),
        in_specs=[
            pl.BlockSpec((tb, D), lambda e, f, t: (t, 0)),
            pl.BlockSpec((tb, 2), lambda e, f, t: (t, 0)),
            pl.BlockSpec((1, 1, fcs, D), lambda e, f, t: (0, e, f, 0)),
            pl.BlockSpec((1, 1, fcs), lambda e, f, t: (e, 0, f)),
            pl.BlockSpec((1, 1, D, fcs), lambda e, f, t: (0, e, 0, f)),
            pl.BlockSpec((1, 1, D), lambda e, f, t: (e, 0, 0)),
            pl.BlockSpec((S, D), lambda e, f, t: (0, 0)),
        ],
        out_specs=[
            pl.BlockSpec((S, D), lambda e, f, t: (0, 0)),
            pl.BlockSpec((1, E), lambda e, f, t: (0, 0)),
            pl.BlockSpec((1, 1), lambda e, f, t: (0, 0)),
        ],
        out_shape=[
            jax.ShapeDtypeStruct((S, D), jnp.float32),
            jax.ShapeDtypeStruct((1, E), jnp.float32),
            jax.ShapeDtypeStruct((1, 1), jnp.float32),
        ],
    )(nx, sel, f1w, f1b, f2w, f2b, hres)


def _lm_body(h_ref, w_ref, b_ref, o_ref, *, tb):
    t = pl.program_id(1)
    x = _bf(h_ref[pl.ds(t * tb, tb), :])
    o_ref[...] = (
        lax.dot_general(x, _bf(w_ref[...]), _CONTRACT_T,
                        preferred_element_type=jnp.float32)
        + b_ref[...]
    )


def _lm_head(h, w, b, tb, vb):
    S, D = h.shape
    V = w.shape[0]
    return pl.pallas_call(
        functools.partial(_lm_body, tb=tb),
        grid=(V // vb, S // tb),
        in_specs=[
            pl.BlockSpec((S, D), lambda v, t: (0, 0)),
            pl.BlockSpec((vb, D), lambda v, t: (v, 0)),
            pl.BlockSpec((1, vb), lambda v, t: (0, v)),
        ],
        out_specs=pl.BlockSpec((tb, vb), lambda v, t: (t, v)),
        out_shape=jax.ShapeDtypeStruct((S, V), jnp.float32),
    )(h, w, b)


def kernel(input_ids, emb, n1w, n1b, n2w, n2b, in_w, in_b, out_w, out_b,
           rw, rb, f1w, f1b, f2w, f2b, lm_w, lm_b):
    Bz, S = input_ids.shape
    V, D = emb.shape
    H = 8
    hd = D // H

    ids = input_ids.reshape(-1).astype(jnp.int32)

    h0 = _embed_gather(emb, ids)

    def _ln(x, w, b):
        m = x.mean(-1, keepdims=True)
        v = x.var(-1, keepdims=True)
        return (x - m) / jnp.sqrt(v + 1e-5) * w + b

    h = h0.reshape(Bz, S, D)
    nx = _ln(h, n1w[0], n1b[0])
    qkv = nx @ in_w[0].T + in_b[0]
    q, k, v = jnp.split(qkv, 3, axis=-1)

    def sp(z):
        return z.reshape(Bz, S, H, hd).transpose(0, 2, 1, 3)

    q, k, v = sp(q), sp(k), sp(v)
    a = jax.nn.softmax(
        (q @ k.transpose(0, 1, 3, 2)) / jnp.sqrt(jnp.float32(hd)), axis=-1)
    o = (a @ v).transpose(0, 2, 1, 3).reshape(Bz, S, D)
    h = h + o @ out_w[0].T + out_b[0]
    nx2 = _ln(h, n2w[0], n2b[0])
    flat = nx2.reshape(-1, D)
    rlog = flat @ rw[0].T + rb[0]
    _, sel = jax.lax.top_k(rlog, 2)

    hfin, _cnt, aux = _moe(flat, sel, f1w, f1b, f2w, f2b,
                           h.reshape(S, D), 512, fchunks=2)
    logits = _lm_head(hfin, lm_w, lm_b.reshape(1, V), 256, 3200)

    return logits.reshape(Bz, S, V), aux[0, 0]
